# baseline (speedup 1.0000x reference)
import functools

import jax
import jax.numpy as jnp
from jax import lax
from jax.experimental import pallas as pl
from jax.experimental.pallas import tpu as pltpu
from jax.experimental.pallas import tpu_sc as plsc

N = 10000
E = 320000
D = 128
G = 64
DOUT = 10

NBLK = 256
N_PAD = 10240
DUMP = N
NT = 32
CHUNK = 128
NCHUNK = 80
E_PAD = NT * CHUNK * NCHUNK
IP = 2
NCH_P = NCHUNK // IP
STRIPE = N_PAD // 16
TOTC = E_PAD // CHUNK
K0 = 122
K1 = (TOTC // 16) - K0


def _sc_mesh():
    return plsc.VectorSubcoreMesh(core_axis_name="c", subcore_axis_name="s")


def _deg_partials(dst2, ones_rows, zrow):
    @functools.partial(
        pl.kernel,
        out_type=jax.ShapeDtypeStruct((2, N_PAD, D), jnp.float32),
        mesh=_sc_mesh(),
        scratch_types=[
            pltpu.VMEM((NCH_P, CHUNK), jnp.int32),
            pltpu.VMEM((CHUNK, D), jnp.float32),
            pltpu.VMEM_SHARED((N_PAD, D), jnp.float32),
        ],
    )
    def k(dst_hbm, ones_hbm, z_hbm, out_hbm, dstv, onesv, acc):
        c = lax.axis_index("c")
        s = lax.axis_index("s")
        wid = c * 16 + s
        pltpu.sync_copy(z_hbm, acc.at[pl.ds(s * STRIPE, STRIPE)])
        pltpu.sync_copy(ones_hbm, onesv)
        plsc.subcore_barrier()

        for p in range(IP):
            st = wid * NCHUNK + p * NCH_P
            pltpu.sync_copy(dst_hbm.at[pl.ds(st, NCH_P)], dstv)

            @pl.loop(0, NCH_P)
            def _(j):
                pltpu.sync_copy(onesv, acc.at[dstv.at[j]], add=True)

        plsc.subcore_barrier()
        pltpu.sync_copy(acc.at[pl.ds(s * STRIPE, STRIPE)],
                        out_hbm.at[c, pl.ds(s * STRIPE, STRIPE)])

    return k(dst2, ones_rows, zrow)


def _agg_partials(g, chunks, zrow):
    @functools.partial(
        pl.kernel,
        out_type=jax.ShapeDtypeStruct((2, N_PAD, D), jnp.float32),
        mesh=_sc_mesh(),
        scratch_types=[
            pltpu.VMEM((2, CHUNK), jnp.int32),
            pltpu.VMEM((2, CHUNK), jnp.int32),
            pltpu.VMEM((CHUNK, D), jnp.float32),
            pltpu.VMEM((CHUNK, D), jnp.float32),
            pltpu.VMEM_SHARED((N_PAD, D), jnp.float32),
            pltpu.SemaphoreType.DMA,
            pltpu.SemaphoreType.DMA,
            pltpu.SemaphoreType.DMA,
            pltpu.SemaphoreType.DMA,
        ],
    )
    def k(g_hbm, idx_hbm, z_hbm, out_hbm,
          ib0, ib1, bufa, bufb, acc, sema, semb, semi0, semi1):
        c = lax.axis_index("c")
        s = lax.axis_index("s")
        start = jnp.where(c == 0, s * K0, 16 * K0 + s * K1)
        npairs = jnp.where(c == 0, K0 // 2, K1 // 2)
        pltpu.sync_copy(z_hbm, acc.at[pl.ds(s * STRIPE, STRIPE)])
        plsc.subcore_barrier()

        pltpu.sync_copy(idx_hbm.at[start], ib0)
        pltpu.async_copy(g_hbm.at[ib0.at[0]], bufa, sema)
        pltpu.async_copy(idx_hbm.at[start + 1], ib1, semi1)

        def pair(p, _):
            j = start + 2 * p
            not_last = p + 1 < npairs
            pltpu.make_async_copy(idx_hbm.at[j + 1], ib1, semi1).wait()
            pltpu.async_copy(g_hbm.at[ib1.at[0]], bufb, semb)
            pltpu.make_async_copy(g_hbm.at[ib0.at[0]], bufa, sema).wait()
            pltpu.sync_copy(bufa, acc.at[ib0.at[1]], add=True)

            @pl.when(not_last)
            def _():
                pltpu.async_copy(idx_hbm.at[j + 2], ib0, semi0)

            pltpu.make_async_copy(g_hbm.at[ib1.at[0]], bufb, semb).wait()
            pltpu.sync_copy(bufb, acc.at[ib1.at[1]], add=True)

            @pl.when(not_last)
            def _():
                pltpu.make_async_copy(idx_hbm.at[j + 2], ib0, semi0).wait()
                pltpu.async_copy(g_hbm.at[ib0.at[0]], bufa, sema)
                pltpu.async_copy(idx_hbm.at[j + 3], ib1, semi1)

            return 0

        lax.fori_loop(0, npairs, pair, 0)

        plsc.subcore_barrier()
        pltpu.sync_copy(acc.at[pl.ds(s * STRIPE, STRIPE)],
                        out_hbm.at[c, pl.ds(s * STRIPE, STRIPE)])

    return k(g, chunks, zrow)


def _dinv_block(d_ref):
    deg = d_ref[0, :, 0:1] + d_ref[1, :, 0:1] + 1.0
    return lax.rsqrt(deg)


def _tc_matmul(x_p, W0):
    def body(x_ref, w_ref, u_ref):
        u_ref[...] = jnp.dot(x_ref[...], w_ref[...],
                             preferred_element_type=jnp.float32)

    return pl.pallas_call(
        body,
        grid=(N_PAD // NBLK,),
        in_specs=[
            pl.BlockSpec((NBLK, D), lambda i: (i, 0)),
            pl.BlockSpec((D, D), lambda i: (0, 0)),
        ],
        out_specs=pl.BlockSpec((NBLK, D), lambda i: (i, 0)),
        out_shape=jax.ShapeDtypeStruct((N_PAD, D), jnp.float32),
    )(x_p, W0)


def _tc_scale(u, degp):
    def body(u_ref, d_ref, g_ref):
        g_ref[...] = _dinv_block(d_ref) * u_ref[...]

    return pl.pallas_call(
        body,
        grid=(N_PAD // NBLK,),
        in_specs=[
            pl.BlockSpec((NBLK, D), lambda i: (i, 0)),
            pl.BlockSpec((2, NBLK, D), lambda i: (0, i, 0)),
        ],
        out_specs=pl.BlockSpec((NBLK, D), lambda i: (i, 0)),
        out_shape=jax.ShapeDtypeStruct((N_PAD, D), jnp.float32),
    )(u, degp)


def _tc_layer(aggp, g_prev, degp, bias, W_next):
    def body(a_ref, g_ref, d_ref, b_ref, w_ref, o_ref):
        dinv = _dinv_block(d_ref)
        h = dinv * (a_ref[0] + a_ref[1] + g_ref[...]) + b_ref[...]
        h = jnp.maximum(h, 0.0)
        o_ref[...] = dinv * jnp.dot(h, w_ref[...],
                                    preferred_element_type=jnp.float32)

    return pl.pallas_call(
        body,
        grid=(N_PAD // NBLK,),
        in_specs=[
            pl.BlockSpec((2, NBLK, D), lambda i: (0, i, 0)),
            pl.BlockSpec((NBLK, D), lambda i: (i, 0)),
            pl.BlockSpec((2, NBLK, D), lambda i: (0, i, 0)),
            pl.BlockSpec((1, D), lambda i: (0, 0)),
            pl.BlockSpec((D, D), lambda i: (0, 0)),
        ],
        out_specs=pl.BlockSpec((NBLK, D), lambda i: (i, 0)),
        out_shape=jax.ShapeDtypeStruct((N_PAD, D), jnp.float32),
    )(aggp, g_prev, degp, bias, W_next)


def _tc_final(aggp, g_prev, degp, bias, batch3, Wlin, blin):
    nsteps = N_PAD // NBLK

    def body(a_ref, g_ref, d_ref, b_ref, bt_ref, wl_ref, bl_ref,
             o_ref, pooled, counts):
        i = pl.program_id(0)

        @pl.when(i == 0)
        def _():
            pooled[...] = jnp.zeros_like(pooled)
            counts[...] = jnp.zeros_like(counts)

        dinv = _dinv_block(d_ref)
        h = dinv * (a_ref[0] + a_ref[1] + g_ref[...]) + b_ref[...]
        h = jnp.maximum(h, 0.0)

        bvals = bt_ref[0]
        gids = lax.broadcasted_iota(jnp.int32, (G, 1), 0)
        onehot = (gids == bvals).astype(jnp.float32)
        pooled[...] += jnp.dot(onehot, h, preferred_element_type=jnp.float32)
        counts[...] += jnp.sum(onehot, axis=1, keepdims=True)

        @pl.when(i == nsteps - 1)
        def _():
            mean = pooled[...] / jnp.maximum(counts[:, 0:1], 1.0)
            o_ref[...] = jnp.dot(mean, wl_ref[...],
                                 preferred_element_type=jnp.float32) + bl_ref[...]

    return pl.pallas_call(
        body,
        grid=(nsteps,),
        in_specs=[
            pl.BlockSpec((2, NBLK, D), lambda i: (0, i, 0)),
            pl.BlockSpec((NBLK, D), lambda i: (i, 0)),
            pl.BlockSpec((2, NBLK, D), lambda i: (0, i, 0)),
            pl.BlockSpec((1, D), lambda i: (0, 0)),
            pl.BlockSpec((1, 1, NBLK), lambda i: (i, 0, 0)),
            pl.BlockSpec((D, DOUT), lambda i: (0, 0)),
            pl.BlockSpec((1, DOUT), lambda i: (0, 0)),
        ],
        out_specs=pl.BlockSpec((G, DOUT), lambda i: (0, 0)),
        out_shape=jax.ShapeDtypeStruct((G, DOUT), jnp.float32),
        scratch_shapes=[
            pltpu.VMEM((G, D), jnp.float32),
            pltpu.VMEM((G, 1), jnp.float32),
        ],
    )(aggp, g_prev, degp, bias, batch3, Wlin, blin)


def kernel(x, edge_index, batch, W0, b0, W, b, Wlin, blin):
    f32 = jnp.float32
    x_p = jnp.zeros((N_PAD, D), f32).at[:N].set(x)

    src = edge_index[0].astype(jnp.int32)
    dst = edge_index[1].astype(jnp.int32)
    padlen = E_PAD - E
    pad_src = jnp.full((padlen,), DUMP, jnp.int32)
    pad_dst = DUMP + (jnp.arange(padlen, dtype=jnp.int32) % (N_PAD - N))
    src_p = jnp.concatenate([src, pad_src])
    dst_p = jnp.concatenate([dst, pad_dst])
    dst2 = dst_p.reshape(TOTC, CHUNK)
    chunks = jnp.stack([src_p.reshape(TOTC, CHUNK), dst2], axis=1)

    batch_p = jnp.full((N_PAD,), G, jnp.int32).at[:N].set(batch.astype(jnp.int32))
    batch3 = batch_p.reshape(N_PAD // NBLK, 1, NBLK)

    ones_rows = jnp.ones((CHUNK, D), f32)
    zrow = jnp.zeros((STRIPE, D), f32)
    b0_2d = b0.reshape(1, D).astype(f32)
    b_2d = b.reshape(1, D).astype(f32)
    blin_2d = blin.reshape(1, DOUT).astype(f32)

    degp = _deg_partials(dst2, ones_rows, zrow)
    u1 = _tc_matmul(x_p, W0.astype(f32))

    g1 = _tc_scale(u1, degp)
    agg1 = _agg_partials(g1, chunks, zrow)
    g2 = _tc_layer(agg1, g1, degp, b0_2d, W.astype(f32))
    agg2 = _agg_partials(g2, chunks, zrow)
    g3 = _tc_layer(agg2, g2, degp, b_2d, W.astype(f32))
    agg3 = _agg_partials(g3, chunks, zrow)

    return _tc_final(agg3, g3, degp, b_2d, batch3, Wlin.astype(f32), blin_2d)


# speedup vs baseline: 1.1150x; 1.1150x over previous
import functools

import jax
import jax.numpy as jnp
from jax import lax
from jax.experimental import pallas as pl
from jax.experimental.pallas import tpu as pltpu
from jax.experimental.pallas import tpu_sc as plsc

N = 10000
E = 320000
D = 128
G = 64
DOUT = 10

NBLK = 256
N_PAD = 10240
DUMP = N
NT = 32
CHUNK = 128
NCHUNK = 80
E_PAD = NT * CHUNK * NCHUNK
IP = 2
NCH_P = NCHUNK // IP
STRIPE = N_PAD // 16
TOTC = E_PAD // CHUNK
K0 = 140
K1 = (TOTC // 16) - K0


def _sc_mesh():
    return plsc.VectorSubcoreMesh(core_axis_name="c", subcore_axis_name="s")


def _deg_partials(dst2, ones_rows, zrow):
    @functools.partial(
        pl.kernel,
        out_type=jax.ShapeDtypeStruct((2, N_PAD, D), jnp.float32),
        mesh=_sc_mesh(),
        scratch_types=[
            pltpu.VMEM((NCH_P, CHUNK), jnp.int32),
            pltpu.VMEM((CHUNK, D), jnp.float32),
            pltpu.VMEM_SHARED((N_PAD, D), jnp.float32),
        ],
    )
    def k(dst_hbm, ones_hbm, z_hbm, out_hbm, dstv, onesv, acc):
        c = lax.axis_index("c")
        s = lax.axis_index("s")
        wid = c * 16 + s
        pltpu.sync_copy(z_hbm, acc.at[pl.ds(s * STRIPE, STRIPE)])
        pltpu.sync_copy(ones_hbm, onesv)
        plsc.subcore_barrier()

        for p in range(IP):
            st = wid * NCHUNK + p * NCH_P
            pltpu.sync_copy(dst_hbm.at[pl.ds(st, NCH_P)], dstv)

            @pl.loop(0, NCH_P)
            def _(j):
                pltpu.sync_copy(onesv, acc.at[dstv.at[j]], add=True)

        plsc.subcore_barrier()
        pltpu.sync_copy(acc.at[pl.ds(s * STRIPE, STRIPE)],
                        out_hbm.at[c, pl.ds(s * STRIPE, STRIPE)])

    return k(dst2, ones_rows, zrow)


def _agg_partials(g, chunks, zrow):
    @functools.partial(
        pl.kernel,
        out_type=jax.ShapeDtypeStruct((2, N_PAD, D), jnp.float32),
        mesh=_sc_mesh(),
        scratch_types=[
            pltpu.VMEM((2, CHUNK), jnp.int32),
            pltpu.VMEM((2, CHUNK), jnp.int32),
            pltpu.VMEM((CHUNK, D), jnp.float32),
            pltpu.VMEM((CHUNK, D), jnp.float32),
            pltpu.VMEM_SHARED((N_PAD, D), jnp.float32),
            pltpu.SemaphoreType.DMA,
            pltpu.SemaphoreType.DMA,
            pltpu.SemaphoreType.DMA,
            pltpu.SemaphoreType.DMA,
        ],
    )
    def k(g_hbm, idx_hbm, z_hbm, out_hbm,
          ib0, ib1, bufa, bufb, acc, sema, semb, semi0, semi1):
        c = lax.axis_index("c")
        s = lax.axis_index("s")
        start = jnp.where(c == 0, s * K0, 16 * K0 + s * K1)
        npairs = jnp.where(c == 0, K0 // 2, K1 // 2)
        pltpu.sync_copy(z_hbm, acc.at[pl.ds(s * STRIPE, STRIPE)])
        plsc.subcore_barrier()

        pltpu.sync_copy(idx_hbm.at[start], ib0)
        pltpu.async_copy(g_hbm.at[ib0.at[0]], bufa, sema)
        pltpu.async_copy(idx_hbm.at[start + 1], ib1, semi1)

        def pair(p, _):
            j = start + 2 * p
            not_last = p + 1 < npairs
            pltpu.make_async_copy(idx_hbm.at[j + 1], ib1, semi1).wait()
            pltpu.async_copy(g_hbm.at[ib1.at[0]], bufb, semb)
            pltpu.make_async_copy(g_hbm.at[ib0.at[0]], bufa, sema).wait()
            pltpu.sync_copy(bufa, acc.at[ib0.at[1]], add=True)

            @pl.when(not_last)
            def _():
                pltpu.async_copy(idx_hbm.at[j + 2], ib0, semi0)

            pltpu.make_async_copy(g_hbm.at[ib1.at[0]], bufb, semb).wait()
            pltpu.sync_copy(bufb, acc.at[ib1.at[1]], add=True)

            @pl.when(not_last)
            def _():
                pltpu.make_async_copy(idx_hbm.at[j + 2], ib0, semi0).wait()
                pltpu.async_copy(g_hbm.at[ib0.at[0]], bufa, sema)
                pltpu.async_copy(idx_hbm.at[j + 3], ib1, semi1)

            return 0

        lax.fori_loop(0, npairs, pair, 0)

        plsc.subcore_barrier()
        pltpu.sync_copy(acc.at[pl.ds(s * STRIPE, STRIPE)],
                        out_hbm.at[c, pl.ds(s * STRIPE, STRIPE)])

    return k(g, chunks, zrow)


def _dinv_block(d_ref):
    deg = d_ref[0, :, 0:1] + d_ref[1, :, 0:1] + 1.0
    return lax.rsqrt(deg)


def _tc_matmul(x_p, W0):
    def body(x_ref, w_ref, u_ref):
        u_ref[...] = jnp.dot(x_ref[...], w_ref[...],
                             preferred_element_type=jnp.float32)

    return pl.pallas_call(
        body,
        grid=(N_PAD // NBLK,),
        in_specs=[
            pl.BlockSpec((NBLK, D), lambda i: (i, 0)),
            pl.BlockSpec((D, D), lambda i: (0, 0)),
        ],
        out_specs=pl.BlockSpec((NBLK, D), lambda i: (i, 0)),
        out_shape=jax.ShapeDtypeStruct((N_PAD, D), jnp.float32),
    )(x_p, W0)


def _tc_scale(u, degp):
    def body(u_ref, d_ref, g_ref):
        g_ref[...] = _dinv_block(d_ref) * u_ref[...]

    return pl.pallas_call(
        body,
        grid=(N_PAD // NBLK,),
        in_specs=[
            pl.BlockSpec((NBLK, D), lambda i: (i, 0)),
            pl.BlockSpec((2, NBLK, D), lambda i: (0, i, 0)),
        ],
        out_specs=pl.BlockSpec((NBLK, D), lambda i: (i, 0)),
        out_shape=jax.ShapeDtypeStruct((N_PAD, D), jnp.float32),
    )(u, degp)


def _tc_layer(aggp, g_prev, degp, bias, W_next):
    def body(a_ref, g_ref, d_ref, b_ref, w_ref, o_ref):
        dinv = _dinv_block(d_ref)
        h = dinv * (a_ref[0] + a_ref[1] + g_ref[...]) + b_ref[...]
        h = jnp.maximum(h, 0.0)
        o_ref[...] = dinv * jnp.dot(h, w_ref[...],
                                    preferred_element_type=jnp.float32)

    return pl.pallas_call(
        body,
        grid=(N_PAD // NBLK,),
        in_specs=[
            pl.BlockSpec((2, NBLK, D), lambda i: (0, i, 0)),
            pl.BlockSpec((NBLK, D), lambda i: (i, 0)),
            pl.BlockSpec((2, NBLK, D), lambda i: (0, i, 0)),
            pl.BlockSpec((1, D), lambda i: (0, 0)),
            pl.BlockSpec((D, D), lambda i: (0, 0)),
        ],
        out_specs=pl.BlockSpec((NBLK, D), lambda i: (i, 0)),
        out_shape=jax.ShapeDtypeStruct((N_PAD, D), jnp.float32),
    )(aggp, g_prev, degp, bias, W_next)


def _tc_final(aggp, g_prev, degp, bias, batch3, Wlin, blin):
    nsteps = N_PAD // NBLK

    def body(a_ref, g_ref, d_ref, b_ref, bt_ref, wl_ref, bl_ref,
             o_ref, pooled, counts):
        i = pl.program_id(0)

        @pl.when(i == 0)
        def _():
            pooled[...] = jnp.zeros_like(pooled)
            counts[...] = jnp.zeros_like(counts)

        dinv = _dinv_block(d_ref)
        h = dinv * (a_ref[0] + a_ref[1] + g_ref[...]) + b_ref[...]
        h = jnp.maximum(h, 0.0)

        bvals = bt_ref[0]
        gids = lax.broadcasted_iota(jnp.int32, (G, 1), 0)
        onehot = (gids == bvals).astype(jnp.float32)
        pooled[...] += jnp.dot(onehot, h, preferred_element_type=jnp.float32)
        counts[...] += jnp.sum(onehot, axis=1, keepdims=True)

        @pl.when(i == nsteps - 1)
        def _():
            mean = pooled[...] / jnp.maximum(counts[:, 0:1], 1.0)
            o_ref[...] = jnp.dot(mean, wl_ref[...],
                                 preferred_element_type=jnp.float32) + bl_ref[...]

    return pl.pallas_call(
        body,
        grid=(nsteps,),
        in_specs=[
            pl.BlockSpec((2, NBLK, D), lambda i: (0, i, 0)),
            pl.BlockSpec((NBLK, D), lambda i: (i, 0)),
            pl.BlockSpec((2, NBLK, D), lambda i: (0, i, 0)),
            pl.BlockSpec((1, D), lambda i: (0, 0)),
            pl.BlockSpec((1, 1, NBLK), lambda i: (i, 0, 0)),
            pl.BlockSpec((D, DOUT), lambda i: (0, 0)),
            pl.BlockSpec((1, DOUT), lambda i: (0, 0)),
        ],
        out_specs=pl.BlockSpec((G, DOUT), lambda i: (0, 0)),
        out_shape=jax.ShapeDtypeStruct((G, DOUT), jnp.float32),
        scratch_shapes=[
            pltpu.VMEM((G, D), jnp.float32),
            pltpu.VMEM((G, 1), jnp.float32),
        ],
    )(aggp, g_prev, degp, bias, batch3, Wlin, blin)


def kernel(x, edge_index, batch, W0, b0, W, b, Wlin, blin):
    f32 = jnp.float32
    x_p = jnp.zeros((N_PAD, D), f32).at[:N].set(x)

    src = edge_index[0].astype(jnp.int32)
    dst = edge_index[1].astype(jnp.int32)
    padlen = E_PAD - E
    pad_src = jnp.full((padlen,), DUMP, jnp.int32)
    pad_dst = DUMP + (jnp.arange(padlen, dtype=jnp.int32) % (N_PAD - N))
    src_p = jnp.concatenate([src, pad_src])
    dst_p = jnp.concatenate([dst, pad_dst])
    dst2 = dst_p.reshape(TOTC, CHUNK)
    chunks = jnp.stack([src_p.reshape(TOTC, CHUNK), dst2], axis=1)

    batch_p = jnp.full((N_PAD,), G, jnp.int32).at[:N].set(batch.astype(jnp.int32))
    batch3 = batch_p.reshape(N_PAD // NBLK, 1, NBLK)

    ones_rows = jnp.ones((CHUNK, D), f32)
    zrow = jnp.zeros((STRIPE, D), f32)
    b0_2d = b0.reshape(1, D).astype(f32)
    b_2d = b.reshape(1, D).astype(f32)
    blin_2d = blin.reshape(1, DOUT).astype(f32)

    degp = _deg_partials(dst2, ones_rows, zrow)
    u1 = _tc_matmul(x_p, W0.astype(f32))

    g1 = _tc_scale(u1, degp)
    agg1 = _agg_partials(g1, chunks, zrow)
    g2 = _tc_layer(agg1, g1, degp, b0_2d, W.astype(f32))
    agg2 = _agg_partials(g2, chunks, zrow)
    g3 = _tc_layer(agg2, g2, degp, b_2d, W.astype(f32))
    agg3 = _agg_partials(g3, chunks, zrow)

    return _tc_final(agg3, g3, degp, b_2d, batch3, Wlin.astype(f32), blin_2d)


# speedup vs baseline: 1.2060x; 1.0816x over previous
import functools

import jax
import jax.numpy as jnp
from jax import lax
from jax.experimental import pallas as pl
from jax.experimental.pallas import tpu as pltpu
from jax.experimental.pallas import tpu_sc as plsc

N = 10000
E = 320000
D = 128
G = 64
DOUT = 10

NBLK = 256
N_PAD = 10240
DUMP = N
NT = 32
CHUNK = 128
NCHUNK = 80
E_PAD = NT * CHUNK * NCHUNK
IP = 2
NCH_P = NCHUNK // IP
STRIPE = N_PAD // 16
TOTC = E_PAD // CHUNK
K0 = 148
K1 = (TOTC // 16) - K0


def _sc_mesh():
    return plsc.VectorSubcoreMesh(core_axis_name="c", subcore_axis_name="s")


def _deg_partials(dst2, ones_rows, zrow):
    @functools.partial(
        pl.kernel,
        out_type=jax.ShapeDtypeStruct((2, N_PAD, D), jnp.float32),
        mesh=_sc_mesh(),
        scratch_types=[
            pltpu.VMEM((NCH_P, CHUNK), jnp.int32),
            pltpu.VMEM((CHUNK, D), jnp.float32),
            pltpu.VMEM_SHARED((N_PAD, D), jnp.float32),
        ],
    )
    def k(dst_hbm, ones_hbm, z_hbm, out_hbm, dstv, onesv, acc):
        c = lax.axis_index("c")
        s = lax.axis_index("s")
        wid = c * 16 + s
        pltpu.sync_copy(z_hbm, acc.at[pl.ds(s * STRIPE, STRIPE)])
        pltpu.sync_copy(ones_hbm, onesv)
        plsc.subcore_barrier()

        for p in range(IP):
            st = wid * NCHUNK + p * NCH_P
            pltpu.sync_copy(dst_hbm.at[pl.ds(st, NCH_P)], dstv)

            @pl.loop(0, NCH_P)
            def _(j):
                pltpu.sync_copy(onesv, acc.at[dstv.at[j]], add=True)

        plsc.subcore_barrier()
        pltpu.sync_copy(acc.at[pl.ds(s * STRIPE, STRIPE)],
                        out_hbm.at[c, pl.ds(s * STRIPE, STRIPE)])

    return k(dst2, ones_rows, zrow)


def _agg_partials(g, chunks, zrow):
    @functools.partial(
        pl.kernel,
        out_type=jax.ShapeDtypeStruct((2, N_PAD, D), jnp.float32),
        mesh=_sc_mesh(),
        scratch_types=[
            pltpu.VMEM((2, CHUNK), jnp.int32),
            pltpu.VMEM((2, CHUNK), jnp.int32),
            pltpu.VMEM((CHUNK, D), jnp.float32),
            pltpu.VMEM((CHUNK, D), jnp.float32),
            pltpu.VMEM_SHARED((N_PAD, D), jnp.float32),
            pltpu.SemaphoreType.DMA,
            pltpu.SemaphoreType.DMA,
            pltpu.SemaphoreType.DMA,
            pltpu.SemaphoreType.DMA,
        ],
    )
    def k(g_hbm, idx_hbm, z_hbm, out_hbm,
          ib0, ib1, bufa, bufb, acc, sema, semb, semi0, semi1):
        c = lax.axis_index("c")
        s = lax.axis_index("s")
        start = jnp.where(c == 0, s * K0, 16 * K0 + s * K1)
        npairs = jnp.where(c == 0, K0 // 2, K1 // 2)
        pltpu.sync_copy(z_hbm, acc.at[pl.ds(s * STRIPE, STRIPE)])
        plsc.subcore_barrier()

        pltpu.sync_copy(idx_hbm.at[start], ib0)
        pltpu.async_copy(g_hbm.at[ib0.at[0]], bufa, sema)
        pltpu.async_copy(idx_hbm.at[start + 1], ib1, semi1)

        def pair(p, _):
            j = start + 2 * p
            not_last = p + 1 < npairs
            pltpu.make_async_copy(idx_hbm.at[j + 1], ib1, semi1).wait()
            pltpu.async_copy(g_hbm.at[ib1.at[0]], bufb, semb)
            pltpu.make_async_copy(g_hbm.at[ib0.at[0]], bufa, sema).wait()
            pltpu.sync_copy(bufa, acc.at[ib0.at[1]], add=True)

            @pl.when(not_last)
            def _():
                pltpu.async_copy(idx_hbm.at[j + 2], ib0, semi0)

            pltpu.make_async_copy(g_hbm.at[ib1.at[0]], bufb, semb).wait()
            pltpu.sync_copy(bufb, acc.at[ib1.at[1]], add=True)

            @pl.when(not_last)
            def _():
                pltpu.make_async_copy(idx_hbm.at[j + 2], ib0, semi0).wait()
                pltpu.async_copy(g_hbm.at[ib0.at[0]], bufa, sema)
                pltpu.async_copy(idx_hbm.at[j + 3], ib1, semi1)

            return 0

        lax.fori_loop(0, npairs, pair, 0)

        plsc.subcore_barrier()
        pltpu.sync_copy(acc.at[pl.ds(s * STRIPE, STRIPE)],
                        out_hbm.at[c, pl.ds(s * STRIPE, STRIPE)])

    return k(g, chunks, zrow)


def _dinv_block(d_ref):
    deg = d_ref[0, :, 0:1] + d_ref[1, :, 0:1] + 1.0
    return lax.rsqrt(deg)


def _tc_matmul(x_p, W0):
    def body(x_ref, w_ref, u_ref):
        u_ref[...] = jnp.dot(x_ref[...], w_ref[...],
                             preferred_element_type=jnp.float32)

    return pl.pallas_call(
        body,
        grid=(N_PAD // NBLK,),
        in_specs=[
            pl.BlockSpec((NBLK, D), lambda i: (i, 0)),
            pl.BlockSpec((D, D), lambda i: (0, 0)),
        ],
        out_specs=pl.BlockSpec((NBLK, D), lambda i: (i, 0)),
        out_shape=jax.ShapeDtypeStruct((N_PAD, D), jnp.float32),
    )(x_p, W0)


def _tc_scale(u, degp):
    def body(u_ref, d_ref, g_ref):
        g_ref[...] = _dinv_block(d_ref) * u_ref[...]

    return pl.pallas_call(
        body,
        grid=(N_PAD // NBLK,),
        in_specs=[
            pl.BlockSpec((NBLK, D), lambda i: (i, 0)),
            pl.BlockSpec((2, NBLK, D), lambda i: (0, i, 0)),
        ],
        out_specs=pl.BlockSpec((NBLK, D), lambda i: (i, 0)),
        out_shape=jax.ShapeDtypeStruct((N_PAD, D), jnp.float32),
    )(u, degp)


def _tc_layer(aggp, g_prev, degp, bias, W_next):
    def body(a_ref, g_ref, d_ref, b_ref, w_ref, o_ref):
        dinv = _dinv_block(d_ref)
        h = dinv * (a_ref[0] + a_ref[1] + g_ref[...]) + b_ref[...]
        h = jnp.maximum(h, 0.0)
        o_ref[...] = dinv * jnp.dot(h, w_ref[...],
                                    preferred_element_type=jnp.float32)

    return pl.pallas_call(
        body,
        grid=(N_PAD // NBLK,),
        in_specs=[
            pl.BlockSpec((2, NBLK, D), lambda i: (0, i, 0)),
            pl.BlockSpec((NBLK, D), lambda i: (i, 0)),
            pl.BlockSpec((2, NBLK, D), lambda i: (0, i, 0)),
            pl.BlockSpec((1, D), lambda i: (0, 0)),
            pl.BlockSpec((D, D), lambda i: (0, 0)),
        ],
        out_specs=pl.BlockSpec((NBLK, D), lambda i: (i, 0)),
        out_shape=jax.ShapeDtypeStruct((N_PAD, D), jnp.float32),
    )(aggp, g_prev, degp, bias, W_next)


def _tc_final(aggp, g_prev, degp, bias, batch3, Wlin, blin):
    nsteps = N_PAD // NBLK

    def body(a_ref, g_ref, d_ref, b_ref, bt_ref, wl_ref, bl_ref,
             o_ref, pooled, counts):
        i = pl.program_id(0)

        @pl.when(i == 0)
        def _():
            pooled[...] = jnp.zeros_like(pooled)
            counts[...] = jnp.zeros_like(counts)

        dinv = _dinv_block(d_ref)
        h = dinv * (a_ref[0] + a_ref[1] + g_ref[...]) + b_ref[...]
        h = jnp.maximum(h, 0.0)

        bvals = bt_ref[0]
        gids = lax.broadcasted_iota(jnp.int32, (G, 1), 0)
        onehot = (gids == bvals).astype(jnp.float32)
        pooled[...] += jnp.dot(onehot, h, preferred_element_type=jnp.float32)
        counts[...] += jnp.sum(onehot, axis=1, keepdims=True)

        @pl.when(i == nsteps - 1)
        def _():
            mean = pooled[...] / jnp.maximum(counts[:, 0:1], 1.0)
            o_ref[...] = jnp.dot(mean, wl_ref[...],
                                 preferred_element_type=jnp.float32) + bl_ref[...]

    return pl.pallas_call(
        body,
        grid=(nsteps,),
        in_specs=[
            pl.BlockSpec((2, NBLK, D), lambda i: (0, i, 0)),
            pl.BlockSpec((NBLK, D), lambda i: (i, 0)),
            pl.BlockSpec((2, NBLK, D), lambda i: (0, i, 0)),
            pl.BlockSpec((1, D), lambda i: (0, 0)),
            pl.BlockSpec((1, 1, NBLK), lambda i: (i, 0, 0)),
            pl.BlockSpec((D, DOUT), lambda i: (0, 0)),
            pl.BlockSpec((1, DOUT), lambda i: (0, 0)),
        ],
        out_specs=pl.BlockSpec((G, DOUT), lambda i: (0, 0)),
        out_shape=jax.ShapeDtypeStruct((G, DOUT), jnp.float32),
        scratch_shapes=[
            pltpu.VMEM((G, D), jnp.float32),
            pltpu.VMEM((G, 1), jnp.float32),
        ],
    )(aggp, g_prev, degp, bias, batch3, Wlin, blin)


def kernel(x, edge_index, batch, W0, b0, W, b, Wlin, blin):
    f32 = jnp.float32
    x_p = jnp.zeros((N_PAD, D), f32).at[:N].set(x)

    src = edge_index[0].astype(jnp.int32)
    dst = edge_index[1].astype(jnp.int32)
    padlen = E_PAD - E
    pad_src = jnp.full((padlen,), DUMP, jnp.int32)
    pad_dst = DUMP + (jnp.arange(padlen, dtype=jnp.int32) % (N_PAD - N))
    src_p = jnp.concatenate([src, pad_src])
    dst_p = jnp.concatenate([dst, pad_dst])
    dst2 = dst_p.reshape(TOTC, CHUNK)
    chunks = jnp.stack([src_p.reshape(TOTC, CHUNK), dst2], axis=1)

    batch_p = jnp.full((N_PAD,), G, jnp.int32).at[:N].set(batch.astype(jnp.int32))
    batch3 = batch_p.reshape(N_PAD // NBLK, 1, NBLK)

    ones_rows = jnp.ones((CHUNK, D), f32)
    zrow = jnp.zeros((STRIPE, D), f32)
    b0_2d = b0.reshape(1, D).astype(f32)
    b_2d = b.reshape(1, D).astype(f32)
    blin_2d = blin.reshape(1, DOUT).astype(f32)

    degp = _deg_partials(dst2, ones_rows, zrow)
    u1 = _tc_matmul(x_p, W0.astype(f32))

    g1 = _tc_scale(u1, degp)
    agg1 = _agg_partials(g1, chunks, zrow)
    g2 = _tc_layer(agg1, g1, degp, b0_2d, W.astype(f32))
    agg2 = _agg_partials(g2, chunks, zrow)
    g3 = _tc_layer(agg2, g2, degp, b_2d, W.astype(f32))
    agg3 = _agg_partials(g3, chunks, zrow)

    return _tc_final(agg3, g3, degp, b_2d, batch3, Wlin.astype(f32), blin_2d)


# speedup vs baseline: 1.2077x; 1.0014x over previous
import functools

import jax
import jax.numpy as jnp
from jax import lax
from jax.experimental import pallas as pl
from jax.experimental.pallas import tpu as pltpu
from jax.experimental.pallas import tpu_sc as plsc

N = 10000
E = 320000
D = 128
G = 64
DOUT = 10

NBLK = 256
N_PAD = 10240
DUMP = N
NT = 32
CHUNK = 128
NCHUNK = 80
E_PAD = NT * CHUNK * NCHUNK
IP = 2
NCH_P = NCHUNK // IP
STRIPE = N_PAD // 16
TOTC = E_PAD // CHUNK
K0 = 154
K1 = (TOTC // 16) - K0


def _sc_mesh():
    return plsc.VectorSubcoreMesh(core_axis_name="c", subcore_axis_name="s")


def _deg_partials(dst2, ones_rows, zrow):
    @functools.partial(
        pl.kernel,
        out_type=jax.ShapeDtypeStruct((2, N_PAD, D), jnp.float32),
        mesh=_sc_mesh(),
        scratch_types=[
            pltpu.VMEM((NCH_P, CHUNK), jnp.int32),
            pltpu.VMEM((CHUNK, D), jnp.float32),
            pltpu.VMEM_SHARED((N_PAD, D), jnp.float32),
        ],
    )
    def k(dst_hbm, ones_hbm, z_hbm, out_hbm, dstv, onesv, acc):
        c = lax.axis_index("c")
        s = lax.axis_index("s")
        wid = c * 16 + s
        pltpu.sync_copy(z_hbm, acc.at[pl.ds(s * STRIPE, STRIPE)])
        pltpu.sync_copy(ones_hbm, onesv)
        plsc.subcore_barrier()

        for p in range(IP):
            st = wid * NCHUNK + p * NCH_P
            pltpu.sync_copy(dst_hbm.at[pl.ds(st, NCH_P)], dstv)

            @pl.loop(0, NCH_P)
            def _(j):
                pltpu.sync_copy(onesv, acc.at[dstv.at[j]], add=True)

        plsc.subcore_barrier()
        pltpu.sync_copy(acc.at[pl.ds(s * STRIPE, STRIPE)],
                        out_hbm.at[c, pl.ds(s * STRIPE, STRIPE)])

    return k(dst2, ones_rows, zrow)


def _agg_partials(g, chunks, zrow):
    @functools.partial(
        pl.kernel,
        out_type=jax.ShapeDtypeStruct((2, N_PAD, D), jnp.float32),
        mesh=_sc_mesh(),
        scratch_types=[
            pltpu.VMEM((2, CHUNK), jnp.int32),
            pltpu.VMEM((2, CHUNK), jnp.int32),
            pltpu.VMEM((CHUNK, D), jnp.float32),
            pltpu.VMEM((CHUNK, D), jnp.float32),
            pltpu.VMEM_SHARED((N_PAD, D), jnp.float32),
            pltpu.SemaphoreType.DMA,
            pltpu.SemaphoreType.DMA,
            pltpu.SemaphoreType.DMA,
            pltpu.SemaphoreType.DMA,
        ],
    )
    def k(g_hbm, idx_hbm, z_hbm, out_hbm,
          ib0, ib1, bufa, bufb, acc, sema, semb, semi0, semi1):
        c = lax.axis_index("c")
        s = lax.axis_index("s")
        start = jnp.where(c == 0, s * K0, 16 * K0 + s * K1)
        npairs = jnp.where(c == 0, K0 // 2, K1 // 2)
        pltpu.sync_copy(z_hbm, acc.at[pl.ds(s * STRIPE, STRIPE)])
        plsc.subcore_barrier()

        pltpu.sync_copy(idx_hbm.at[start], ib0)
        pltpu.async_copy(g_hbm.at[ib0.at[0]], bufa, sema)
        pltpu.async_copy(idx_hbm.at[start + 1], ib1, semi1)

        def pair(p, _):
            j = start + 2 * p
            not_last = p + 1 < npairs
            pltpu.make_async_copy(idx_hbm.at[j + 1], ib1, semi1).wait()
            pltpu.async_copy(g_hbm.at[ib1.at[0]], bufb, semb)
            pltpu.make_async_copy(g_hbm.at[ib0.at[0]], bufa, sema).wait()
            pltpu.sync_copy(bufa, acc.at[ib0.at[1]], add=True)

            @pl.when(not_last)
            def _():
                pltpu.async_copy(idx_hbm.at[j + 2], ib0, semi0)

            pltpu.make_async_copy(g_hbm.at[ib1.at[0]], bufb, semb).wait()
            pltpu.sync_copy(bufb, acc.at[ib1.at[1]], add=True)

            @pl.when(not_last)
            def _():
                pltpu.make_async_copy(idx_hbm.at[j + 2], ib0, semi0).wait()
                pltpu.async_copy(g_hbm.at[ib0.at[0]], bufa, sema)
                pltpu.async_copy(idx_hbm.at[j + 3], ib1, semi1)

            return 0

        lax.fori_loop(0, npairs, pair, 0)

        plsc.subcore_barrier()
        pltpu.sync_copy(acc.at[pl.ds(s * STRIPE, STRIPE)],
                        out_hbm.at[c, pl.ds(s * STRIPE, STRIPE)])

    return k(g, chunks, zrow)


def _dinv_block(d_ref):
    deg = d_ref[0, :, 0:1] + d_ref[1, :, 0:1] + 1.0
    return lax.rsqrt(deg)


def _tc_matmul(x_p, W0):
    def body(x_ref, w_ref, u_ref):
        u_ref[...] = jnp.dot(x_ref[...], w_ref[...],
                             preferred_element_type=jnp.float32)

    return pl.pallas_call(
        body,
        grid=(N_PAD // NBLK,),
        in_specs=[
            pl.BlockSpec((NBLK, D), lambda i: (i, 0)),
            pl.BlockSpec((D, D), lambda i: (0, 0)),
        ],
        out_specs=pl.BlockSpec((NBLK, D), lambda i: (i, 0)),
        out_shape=jax.ShapeDtypeStruct((N_PAD, D), jnp.float32),
    )(x_p, W0)


def _tc_scale(u, degp):
    def body(u_ref, d_ref, g_ref):
        g_ref[...] = _dinv_block(d_ref) * u_ref[...]

    return pl.pallas_call(
        body,
        grid=(N_PAD // NBLK,),
        in_specs=[
            pl.BlockSpec((NBLK, D), lambda i: (i, 0)),
            pl.BlockSpec((2, NBLK, D), lambda i: (0, i, 0)),
        ],
        out_specs=pl.BlockSpec((NBLK, D), lambda i: (i, 0)),
        out_shape=jax.ShapeDtypeStruct((N_PAD, D), jnp.float32),
    )(u, degp)


def _tc_layer(aggp, g_prev, degp, bias, W_next):
    def body(a_ref, g_ref, d_ref, b_ref, w_ref, o_ref):
        dinv = _dinv_block(d_ref)
        h = dinv * (a_ref[0] + a_ref[1] + g_ref[...]) + b_ref[...]
        h = jnp.maximum(h, 0.0)
        o_ref[...] = dinv * jnp.dot(h, w_ref[...],
                                    preferred_element_type=jnp.float32)

    return pl.pallas_call(
        body,
        grid=(N_PAD // NBLK,),
        in_specs=[
            pl.BlockSpec((2, NBLK, D), lambda i: (0, i, 0)),
            pl.BlockSpec((NBLK, D), lambda i: (i, 0)),
            pl.BlockSpec((2, NBLK, D), lambda i: (0, i, 0)),
            pl.BlockSpec((1, D), lambda i: (0, 0)),
            pl.BlockSpec((D, D), lambda i: (0, 0)),
        ],
        out_specs=pl.BlockSpec((NBLK, D), lambda i: (i, 0)),
        out_shape=jax.ShapeDtypeStruct((N_PAD, D), jnp.float32),
    )(aggp, g_prev, degp, bias, W_next)


def _tc_final(aggp, g_prev, degp, bias, batch3, Wlin, blin):
    nsteps = N_PAD // NBLK

    def body(a_ref, g_ref, d_ref, b_ref, bt_ref, wl_ref, bl_ref,
             o_ref, pooled, counts):
        i = pl.program_id(0)

        @pl.when(i == 0)
        def _():
            pooled[...] = jnp.zeros_like(pooled)
            counts[...] = jnp.zeros_like(counts)

        dinv = _dinv_block(d_ref)
        h = dinv * (a_ref[0] + a_ref[1] + g_ref[...]) + b_ref[...]
        h = jnp.maximum(h, 0.0)

        bvals = bt_ref[0]
        gids = lax.broadcasted_iota(jnp.int32, (G, 1), 0)
        onehot = (gids == bvals).astype(jnp.float32)
        pooled[...] += jnp.dot(onehot, h, preferred_element_type=jnp.float32)
        counts[...] += jnp.sum(onehot, axis=1, keepdims=True)

        @pl.when(i == nsteps - 1)
        def _():
            mean = pooled[...] / jnp.maximum(counts[:, 0:1], 1.0)
            o_ref[...] = jnp.dot(mean, wl_ref[...],
                                 preferred_element_type=jnp.float32) + bl_ref[...]

    return pl.pallas_call(
        body,
        grid=(nsteps,),
        in_specs=[
            pl.BlockSpec((2, NBLK, D), lambda i: (0, i, 0)),
            pl.BlockSpec((NBLK, D), lambda i: (i, 0)),
            pl.BlockSpec((2, NBLK, D), lambda i: (0, i, 0)),
            pl.BlockSpec((1, D), lambda i: (0, 0)),
            pl.BlockSpec((1, 1, NBLK), lambda i: (i, 0, 0)),
            pl.BlockSpec((D, DOUT), lambda i: (0, 0)),
            pl.BlockSpec((1, DOUT), lambda i: (0, 0)),
        ],
        out_specs=pl.BlockSpec((G, DOUT), lambda i: (0, 0)),
        out_shape=jax.ShapeDtypeStruct((G, DOUT), jnp.float32),
        scratch_shapes=[
            pltpu.VMEM((G, D), jnp.float32),
            pltpu.VMEM((G, 1), jnp.float32),
        ],
    )(aggp, g_prev, degp, bias, batch3, Wlin, blin)


def kernel(x, edge_index, batch, W0, b0, W, b, Wlin, blin):
    f32 = jnp.float32
    x_p = jnp.zeros((N_PAD, D), f32).at[:N].set(x)

    src = edge_index[0].astype(jnp.int32)
    dst = edge_index[1].astype(jnp.int32)
    padlen = E_PAD - E
    pad_src = jnp.full((padlen,), DUMP, jnp.int32)
    pad_dst = DUMP + (jnp.arange(padlen, dtype=jnp.int32) % (N_PAD - N))
    src_p = jnp.concatenate([src, pad_src])
    dst_p = jnp.concatenate([dst, pad_dst])
    dst2 = dst_p.reshape(TOTC, CHUNK)
    chunks = jnp.stack([src_p.reshape(TOTC, CHUNK), dst2], axis=1)

    batch_p = jnp.full((N_PAD,), G, jnp.int32).at[:N].set(batch.astype(jnp.int32))
    batch3 = batch_p.reshape(N_PAD // NBLK, 1, NBLK)

    ones_rows = jnp.ones((CHUNK, D), f32)
    zrow = jnp.zeros((STRIPE, D), f32)
    b0_2d = b0.reshape(1, D).astype(f32)
    b_2d = b.reshape(1, D).astype(f32)
    blin_2d = blin.reshape(1, DOUT).astype(f32)

    degp = _deg_partials(dst2, ones_rows, zrow)
    u1 = _tc_matmul(x_p, W0.astype(f32))

    g1 = _tc_scale(u1, degp)
    agg1 = _agg_partials(g1, chunks, zrow)
    g2 = _tc_layer(agg1, g1, degp, b0_2d, W.astype(f32))
    agg2 = _agg_partials(g2, chunks, zrow)
    g3 = _tc_layer(agg2, g2, degp, b_2d, W.astype(f32))
    agg3 = _agg_partials(g3, chunks, zrow)

    return _tc_final(agg3, g3, degp, b_2d, batch3, Wlin.astype(f32), blin_2d)
